# baseline (speedup 1.0000x reference)
import functools

import jax
import jax.numpy as jnp
import numpy as np
from jax import lax
from jax.experimental import pallas as pl
from jax.experimental.pallas import tpu as pltpu
from jax.experimental.pallas import tpu_sc as plsc

_RRF_KCONST = 60.0
_EMPTY = np.int32(-1)
_H = 2048
_HSH = np.int32(32 - 11)
_L = 16
_NW = 32
_HASH_MULT = np.int32(-1640531527)


def _bubble5(acc, cs, cd):
    (s0, s1, s2, s3, s4, d0, d1, d2, d3, d4) = acc
    new = []
    for si, di in ((s0, d0), (s1, d1), (s2, d2), (s3, d3), (s4, d4)):
        better = (cs > si) | ((cs == si) & (cd < di))
        ns = jnp.where(better, cs, si)
        nd = jnp.where(better, cd, di)
        cs = jnp.where(better, si, cs)
        cd = jnp.where(better, di, cd)
        new.append((ns, nd))
    return (new[0][0], new[1][0], new[2][0], new[3][0], new[4][0],
            new[0][1], new[1][1], new[2][1], new[3][1], new[4][1])


def _fuse_body(ids_hbm, pos_hbm, sc_hbm, out_hbm,
               blk_rm0, blk_rm1, blk, tid, ts, srec, ovf, cand_s, cand_d,
               sc_v, pos_v, outb0, outb1, sem0, sem1, semo):
    N = sc_hbm.shape[0]
    B = pos_hbm.shape[0]
    rows_per_w = B // _NW
    groups = rows_per_w // _L
    wid = lax.axis_index("s") * 2 + lax.axis_index("c")
    lane = lax.iota(jnp.int32, _L)
    blk_rms = (blk_rm0, blk_rm1)
    outbs = (outb0, outb1)
    sems = (sem0, sem1)

    cps = []
    for g in range(groups):
        base = wid * rows_per_w + g * _L
        cps.append(pltpu.async_copy(
            ids_hbm.at[pl.ds(base * np.int32(N), N * _L)],
            blk_rms[g], sems[g]))

    pltpu.sync_copy(sc_hbm, sc_v)

    @plsc.parallel_loop(0, _H, unroll=8)
    def _clear(h):
        tid[pl.ds(h * _L, _L)] = jnp.full((_L,), _EMPTY, jnp.int32)

    zf = jnp.zeros((_L,), jnp.float32)
    zi = jnp.zeros((_L,), jnp.int32)
    neg1 = jnp.full((_L,), -1.0, jnp.float32)
    emptyv = jnp.full((_L,), _EMPTY, jnp.int32)
    hmask = np.int32(_H - 1)
    out_cp = None

    for g in range(groups):
        base = wid * rows_per_w + g * _L
        blk_rm = blk_rms[g]
        outb = outbs[g]
        cps[g].wait()
        pltpu.sync_copy(pos_hbm.at[pl.ds(base, _L)], pos_v)

        lane_n = lane * np.int32(N)

        @plsc.parallel_loop(0, N, unroll=16)
        def _tr(j):
            rot = (lane + (j & np.int32(_L - 1))) & np.int32(_L - 1)
            jd = (j & np.int32(~(_L - 1))) + rot
            v = plsc.load_gather(blk_rm, [lane_n + jd])
            plsc.store_scatter(blk, [jd * np.int32(_L) + lane], v)

        def build_body(t, ocnt):
            offs = [t * np.int32(4 * _L) + np.int32(u * _L) for u in range(4)]
            jbase = t * np.int32(4)
            vids = [blk[pl.ds(o, _L)] for o in offs]
            sjs = [plsc.load_gather(
                sc_v, [jnp.full((_L,), jbase + np.int32(u), jnp.int32)])
                for u in range(4)]
            s1s, s2s, st1s, st2s = [], [], [], []
            for u in range(4):
                h = lax.shift_right_logical(vids[u] * _HASH_MULT, _HSH)
                s1s.append(h * np.int32(_L) + lane)
                s2s.append((((h + 1) & hmask) * np.int32(_L)) + lane)
            for u in range(4):
                st1s.append(plsc.load_gather(tid, [s1s[u]]))
                st2s.append(plsc.load_gather(tid, [s2s[u]]))

            ms, es, slots, overs = [], [], [], []
            for u in range(4):
                vid = vids[u]
                m1 = st1s[u] == vid
                e1 = st1s[u] == _EMPTY
                m2 = st2s[u] == vid
                e2 = st2s[u] == _EMPTY
                for q in range(u):
                    cl = es[q]
                    e1 = e1 & ~(cl & (s1s[u] == slots[q]))
                    e2 = e2 & ~(cl & (s2s[u] == slots[q]))
                hit1 = m1 | e1
                m_own = m1 | ((~hit1) & m2)
                e_own = e1 | ((~hit1) & e2)
                slot_own = jnp.where(hit1, s1s[u], s2s[u])
                if u == 0:
                    m_u, e_u, slot_u = m_own, e_own, slot_own
                else:
                    same_any = jnp.zeros((_L,), jnp.bool_)
                    follow_hit = jnp.zeros((_L,), jnp.bool_)
                    follow_slot = slot_own
                    for q in range(u):
                        same_q = vid == vids[q]
                        hit_q = ms[q] | es[q]
                        same_any = same_any | same_q
                        follow_slot = jnp.where(same_q & hit_q,
                                                slots[q], follow_slot)
                        follow_hit = follow_hit | (same_q & hit_q)
                    m_u = ((~same_any) & m_own) | follow_hit
                    e_u = (~same_any) & e_own
                    slot_u = jnp.where(follow_hit, follow_slot, slot_own)
                ms.append(m_u)
                es.append(e_u)
                slots.append(slot_u)
                overs.append(~(m_u | e_u))

            for u in range(4):
                plsc.store_scatter(tid, [slots[u]], vids[u], mask=es[u])
                plsc.store_scatter(ts, [slots[u]], sjs[u], mask=es[u])
                plsc.addupdate_scatter(ts, [slots[u]], sjs[u], mask=ms[u])
                srec[pl.ds(offs[u], _L)] = jnp.where(es[u], slots[u], _EMPTY)
            for u in range(4):
                plsc.store_scatter(
                    ovf, [ocnt * np.int32(_L) + lane],
                    jnp.full((_L,), jbase + np.int32(u), jnp.int32),
                    mask=overs[u])
                ocnt = ocnt + jnp.where(overs[u], 1, 0)
            return ocnt

        ocnt = lax.fori_loop(0, N // 4, build_body, zi)

        def ocond(carry):
            k, _h, _f = carry
            return jnp.any(k < ocnt)

        def obody(carry):
            k, h, fresh = carry
            active = k < ocnt
            jv = plsc.load_gather(ovf, [k * np.int32(_L) + lane], mask=active)
            ioff = jv * np.int32(_L) + lane
            vid = plsc.load_gather(blk, [ioff], mask=active)
            sj = plsc.load_gather(sc_v, [jv], mask=active)
            h = jnp.where(fresh,
                          lax.shift_right_logical(vid * _HASH_MULT, _HSH), h)
            slot = h * np.int32(_L) + lane
            stored = plsc.load_gather(tid, [slot], mask=active)
            is_match = active & (stored == vid)
            is_empty = active & (stored == _EMPTY)
            hit = is_match | is_empty
            plsc.store_scatter(tid, [slot], vid, mask=is_empty)
            plsc.store_scatter(ts, [slot], sj, mask=is_empty)
            plsc.addupdate_scatter(ts, [slot], sj, mask=is_match)
            plsc.store_scatter(srec, [ioff], slot, mask=is_empty)
            return (k + jnp.where(hit, 1, 0),
                    jnp.where(hit, h, (h + 1) & hmask),
                    hit)

        lax.while_loop(ocond, obody, (zi, zi, jnp.ones((_L,), jnp.bool_)))

        init = (zf, zf, zf, zf, zf, zi, zi, zi, zi, zi)

        acc0 = init
        k0 = N // 4
        for j in (0, 1, k0, k0 + 1, 2 * k0, 2 * k0 + 1,
                  3 * k0, 3 * k0 + 1):
            if j >= N:
                continue
            off = j * _L
            slot = srec[pl.ds(off, _L)]
            cd = blk[pl.ds(off, _L)]
            first = slot >= 0
            cs = plsc.load_gather(ts, [slot], mask=first)
            cs = jnp.where(first, cs, neg1)
            acc0 = _bubble5(acc0, cs, cd)
        s4 = acc0[4]
        d4 = acc0[9]

        @plsc.parallel_loop(0, N, unroll=8, carry=zi)
        def _filt(j, cnt):
            off = j * np.int32(_L)
            slot = srec[pl.ds(off, _L)]
            cd = blk[pl.ds(off, _L)]
            first = slot >= 0
            cs = plsc.load_gather(ts, [slot], mask=first)
            cs = jnp.where(first, cs, neg1)
            geq = (cs > s4) | ((cs == s4) & (cd <= d4))
            idx = cnt * np.int32(_L) + lane
            plsc.store_scatter(cand_s, [idx], cs, mask=geq)
            plsc.store_scatter(cand_d, [idx], cd, mask=geq)
            return cnt + jnp.where(geq, 1, 0)

        ccnt = _filt
        maxc = lax.reduce_max(ccnt, (0,))

        def fin_body(c, acc):
            cv = jnp.full((_L,), c, jnp.int32)
            active = cv < ccnt
            idx = cv * np.int32(_L) + lane
            s = plsc.load_gather(cand_s, [idx], mask=active)
            d = plsc.load_gather(cand_d, [idx], mask=active)
            s = jnp.where(active, s, neg1)
            return _bubble5(acc, s, d)

        acc_a = lax.fori_loop(0, maxc, fin_body, init)
        d_top = acc_a[5:]

        if g + 1 < groups:
            @plsc.parallel_loop(0, N, unroll=4)
            def _rc(j):
                slot = srec[pl.ds(j * np.int32(_L), _L)]
                plsc.store_scatter(tid, [slot], emptyv, mask=slot >= 0)

        p = pos_v[:]
        res = d_top[0]
        for i in range(1, 5):
            res = jnp.where(p == np.int32(i), d_top[i], res)
        outb[:] = res
        out_cp = pltpu.async_copy(outb, out_hbm.at[pl.ds(base, _L)], semo)
        if g + 1 == groups:
            out_cp.wait()
        else:
            prev_out_cp = out_cp

    if groups > 1:
        prev_out_cp.wait()


def kernel(index_batch, positions, weight):
    B, T, K = index_batch.shape
    N = T * K
    rank = jnp.arange(1, K + 1, dtype=jnp.float32)
    teacher_w = weight[:T][:, None]
    slot_scores = (teacher_w / (_RRF_KCONST + rank[None, :])).reshape(-1)
    ids_flat = index_batch.reshape(B * N)

    run = functools.partial(
        pl.kernel,
        out_type=jax.ShapeDtypeStruct((B,), jnp.int32),
        mesh=plsc.VectorSubcoreMesh(core_axis_name="c", subcore_axis_name="s"),
        compiler_params=pltpu.CompilerParams(
            needs_layout_passes=False,
            disable_bounds_checks=True,
            disable_semaphore_checks=True,
        ),
        scratch_types=[
            pltpu.VMEM((N * _L,), jnp.int32),
            pltpu.VMEM((N * _L,), jnp.int32),
            pltpu.VMEM((N * _L,), jnp.int32),
            pltpu.VMEM((_H * _L,), jnp.int32),
            pltpu.VMEM((_H * _L,), jnp.float32),
            pltpu.VMEM((N * _L,), jnp.int32),
            pltpu.VMEM((N * _L,), jnp.int32),
            pltpu.VMEM((N * _L,), jnp.float32),
            pltpu.VMEM((N * _L,), jnp.int32),
            pltpu.VMEM((N,), jnp.float32),
            pltpu.VMEM((_L,), jnp.int32),
            pltpu.VMEM((_L,), jnp.int32),
            pltpu.VMEM((_L,), jnp.int32),
            pltpu.SemaphoreType.DMA,
            pltpu.SemaphoreType.DMA,
            pltpu.SemaphoreType.DMA,
        ],
    )(_fuse_body)
    return run(ids_flat, positions, slot_scores)


# speedup vs baseline: 1.0394x; 1.0394x over previous
import functools

import jax
import jax.numpy as jnp
import numpy as np
from jax import lax
from jax.experimental import pallas as pl
from jax.experimental.pallas import tpu as pltpu
from jax.experimental.pallas import tpu_sc as plsc

_RRF_KCONST = 60.0
_EMPTY = np.int32(-1)
_H = 2048
_HSH = np.int32(32 - 11)
_L = 16
_NW = 32
_HASH_MULT = np.int32(-1640531527)


def _bubble5(acc, cs, cd):
    (s0, s1, s2, s3, s4, d0, d1, d2, d3, d4) = acc
    new = []
    for si, di in ((s0, d0), (s1, d1), (s2, d2), (s3, d3), (s4, d4)):
        better = (cs > si) | ((cs == si) & (cd < di))
        ns = jnp.where(better, cs, si)
        nd = jnp.where(better, cd, di)
        cs = jnp.where(better, si, cs)
        cd = jnp.where(better, di, cd)
        new.append((ns, nd))
    return (new[0][0], new[1][0], new[2][0], new[3][0], new[4][0],
            new[0][1], new[1][1], new[2][1], new[3][1], new[4][1])


def _fuse_body(ids_hbm, pos_hbm, sc_hbm, out_hbm,
               blk_rm0, blk_rm1, blk, tid, ts, srec, ovf, cand_s, cand_d,
               sc_v, pos_v, outb0, outb1, sem0, sem1, semo):
    N = sc_hbm.shape[0]
    B = pos_hbm.shape[0]
    rows_per_w = B // _NW
    groups = rows_per_w // _L
    wid = lax.axis_index("s") * 2 + lax.axis_index("c")
    lane = lax.iota(jnp.int32, _L)
    blk_rms = (blk_rm0, blk_rm1)
    outbs = (outb0, outb1)
    sems = (sem0, sem1)

    cps = []
    for g in range(groups):
        base = wid * rows_per_w + g * _L
        cps.append(pltpu.async_copy(
            ids_hbm.at[pl.ds(base * np.int32(N), N * _L)],
            blk_rms[g], sems[g]))

    pltpu.sync_copy(sc_hbm, sc_v)

    @plsc.parallel_loop(0, _H, unroll=8)
    def _clear(h):
        tid[pl.ds(h * _L, _L)] = jnp.full((_L,), _EMPTY, jnp.int32)

    zf = jnp.zeros((_L,), jnp.float32)
    zi = jnp.zeros((_L,), jnp.int32)
    neg1 = jnp.full((_L,), -1.0, jnp.float32)
    emptyv = jnp.full((_L,), _EMPTY, jnp.int32)
    hmask = np.int32(_H - 1)
    out_cp = None

    for g in range(groups):
        base = wid * rows_per_w + g * _L
        blk_rm = blk_rms[g]
        outb = outbs[g]
        cps[g].wait()
        pltpu.sync_copy(pos_hbm.at[pl.ds(base, _L)], pos_v)

        lane_n = lane * np.int32(N)

        @plsc.parallel_loop(0, N, unroll=16)
        def _tr(j):
            rot = (lane + (j & np.int32(_L - 1))) & np.int32(_L - 1)
            jd = (j & np.int32(~(_L - 1))) + rot
            v = plsc.load_gather(blk_rm, [lane_n + jd])
            plsc.store_scatter(blk, [jd * np.int32(_L) + lane], v)

        def build_body(t, ocnt):
            offs = [t * np.int32(4 * _L) + np.int32(u * _L) for u in range(4)]
            jbase = t * np.int32(4)
            vids = [blk[pl.ds(o, _L)] for o in offs]
            sjs = [plsc.load_gather(
                sc_v, [jnp.full((_L,), jbase + np.int32(u), jnp.int32)])
                for u in range(4)]
            s1s, s2s, st1s, st2s = [], [], [], []
            for u in range(4):
                h = lax.shift_right_logical(vids[u] * _HASH_MULT, _HSH)
                s1s.append(h * np.int32(_L) + lane)
                s2s.append((((h + 1) & hmask) * np.int32(_L)) + lane)
            for u in range(4):
                st1s.append(plsc.load_gather(tid, [s1s[u]]))
                st2s.append(plsc.load_gather(tid, [s2s[u]]))

            ms, es, slots, overs = [], [], [], []
            for u in range(4):
                vid = vids[u]
                m1 = st1s[u] == vid
                e1 = st1s[u] == _EMPTY
                m2 = st2s[u] == vid
                e2 = st2s[u] == _EMPTY
                for q in range(u):
                    cl = es[q]
                    e1 = e1 & ~(cl & (s1s[u] == slots[q]))
                    e2 = e2 & ~(cl & (s2s[u] == slots[q]))
                hit1 = m1 | e1
                m_own = m1 | ((~hit1) & m2)
                e_own = e1 | ((~hit1) & e2)
                slot_own = jnp.where(hit1, s1s[u], s2s[u])
                if u == 0:
                    m_u, e_u, slot_u = m_own, e_own, slot_own
                else:
                    same_any = jnp.zeros((_L,), jnp.bool_)
                    follow_hit = jnp.zeros((_L,), jnp.bool_)
                    follow_slot = slot_own
                    for q in range(u):
                        same_q = vid == vids[q]
                        hit_q = ms[q] | es[q]
                        same_any = same_any | same_q
                        follow_slot = jnp.where(same_q & hit_q,
                                                slots[q], follow_slot)
                        follow_hit = follow_hit | (same_q & hit_q)
                    m_u = ((~same_any) & m_own) | follow_hit
                    e_u = (~same_any) & e_own
                    slot_u = jnp.where(follow_hit, follow_slot, slot_own)
                ms.append(m_u)
                es.append(e_u)
                slots.append(slot_u)
                overs.append(~(m_u | e_u))

            for u in range(4):
                plsc.store_scatter(tid, [slots[u]], vids[u], mask=es[u])
                plsc.store_scatter(ts, [slots[u]], sjs[u], mask=es[u])
                plsc.addupdate_scatter(ts, [slots[u]], sjs[u], mask=ms[u])
                srec[pl.ds(offs[u], _L)] = jnp.where(es[u], slots[u], _EMPTY)
            for u in range(4):
                plsc.store_scatter(
                    ovf, [ocnt * np.int32(_L) + lane],
                    jnp.full((_L,), jbase + np.int32(u), jnp.int32),
                    mask=overs[u])
                ocnt = ocnt + jnp.where(overs[u], 1, 0)
            return ocnt

        ocnt = lax.fori_loop(0, N // 4, build_body, zi)

        def ocond(carry):
            k, _h, _f = carry
            return jnp.any(k < ocnt)

        def obody(carry):
            k, h, fresh = carry
            active = k < ocnt
            jv = plsc.load_gather(ovf, [k * np.int32(_L) + lane], mask=active)
            ioff = jv * np.int32(_L) + lane
            vid = plsc.load_gather(blk, [ioff], mask=active)
            sj = plsc.load_gather(sc_v, [jv], mask=active)
            h = jnp.where(fresh,
                          lax.shift_right_logical(vid * _HASH_MULT, _HSH), h)
            slot = h * np.int32(_L) + lane
            stored = plsc.load_gather(tid, [slot], mask=active)
            is_match = active & (stored == vid)
            is_empty = active & (stored == _EMPTY)
            hit = is_match | is_empty
            plsc.store_scatter(tid, [slot], vid, mask=is_empty)
            plsc.store_scatter(ts, [slot], sj, mask=is_empty)
            plsc.addupdate_scatter(ts, [slot], sj, mask=is_match)
            plsc.store_scatter(srec, [ioff], slot, mask=is_empty)
            return (k + jnp.where(hit, 1, 0),
                    jnp.where(hit, h, (h + 1) & hmask),
                    hit)

        lax.while_loop(ocond, obody, (zi, zi, jnp.ones((_L,), jnp.bool_)))

        init = (zf, zf, zf, zf, zf, zi, zi, zi, zi, zi)

        acc0 = init
        k0 = N // 4
        for j in (0, 1, k0, k0 + 1, 2 * k0, 2 * k0 + 1,
                  3 * k0, 3 * k0 + 1):
            if j >= N:
                continue
            off = j * _L
            slot = srec[pl.ds(off, _L)]
            cd = blk[pl.ds(off, _L)]
            first = slot >= 0
            cs = plsc.load_gather(ts, [slot], mask=first)
            cs = jnp.where(first, cs, neg1)
            acc0 = _bubble5(acc0, cs, cd)
        s4 = acc0[4]
        d4 = acc0[9]

        @plsc.parallel_loop(0, N, unroll=4, carry=zi)
        def _filt(j, cnt):
            off = j * np.int32(_L)
            slot = srec[pl.ds(off, _L)]
            cd = blk[pl.ds(off, _L)]
            first = slot >= 0
            cs = plsc.load_gather(ts, [slot], mask=first)
            cs = jnp.where(first, cs, neg1)
            geq = (cs > s4) | ((cs == s4) & (cd <= d4))
            idx = cnt * np.int32(_L) + lane
            plsc.store_scatter(cand_s, [idx], cs, mask=geq)
            plsc.store_scatter(cand_d, [idx], cd, mask=geq)
            return cnt + jnp.where(geq, 1, 0)

        ccnt = _filt
        maxc = lax.reduce_max(ccnt, (0,))

        def fin_body(c, acc):
            cv = jnp.full((_L,), c, jnp.int32)
            active = cv < ccnt
            idx = cv * np.int32(_L) + lane
            s = plsc.load_gather(cand_s, [idx], mask=active)
            d = plsc.load_gather(cand_d, [idx], mask=active)
            s = jnp.where(active, s, neg1)
            return _bubble5(acc, s, d)

        acc_a = lax.fori_loop(0, maxc, fin_body, init)
        d_top = acc_a[5:]

        if g + 1 < groups:
            @plsc.parallel_loop(0, N, unroll=4)
            def _rc(j):
                slot = srec[pl.ds(j * np.int32(_L), _L)]
                plsc.store_scatter(tid, [slot], emptyv, mask=slot >= 0)

        p = pos_v[:]
        res = d_top[0]
        for i in range(1, 5):
            res = jnp.where(p == np.int32(i), d_top[i], res)
        outb[:] = res
        out_cp = pltpu.async_copy(outb, out_hbm.at[pl.ds(base, _L)], semo)
        if g + 1 == groups:
            out_cp.wait()
        else:
            prev_out_cp = out_cp

    if groups > 1:
        prev_out_cp.wait()


def kernel(index_batch, positions, weight):
    B, T, K = index_batch.shape
    N = T * K
    rank = jnp.arange(1, K + 1, dtype=jnp.float32)
    teacher_w = weight[:T][:, None]
    slot_scores = (teacher_w / (_RRF_KCONST + rank[None, :])).reshape(-1)
    ids_flat = index_batch.reshape(B * N)

    run = functools.partial(
        pl.kernel,
        out_type=jax.ShapeDtypeStruct((B,), jnp.int32),
        mesh=plsc.VectorSubcoreMesh(core_axis_name="c", subcore_axis_name="s"),
        compiler_params=pltpu.CompilerParams(needs_layout_passes=False),
        scratch_types=[
            pltpu.VMEM((N * _L,), jnp.int32),
            pltpu.VMEM((N * _L,), jnp.int32),
            pltpu.VMEM((N * _L,), jnp.int32),
            pltpu.VMEM((_H * _L,), jnp.int32),
            pltpu.VMEM((_H * _L,), jnp.float32),
            pltpu.VMEM((N * _L,), jnp.int32),
            pltpu.VMEM((N * _L,), jnp.int32),
            pltpu.VMEM((N * _L,), jnp.float32),
            pltpu.VMEM((N * _L,), jnp.int32),
            pltpu.VMEM((N,), jnp.float32),
            pltpu.VMEM((_L,), jnp.int32),
            pltpu.VMEM((_L,), jnp.int32),
            pltpu.VMEM((_L,), jnp.int32),
            pltpu.SemaphoreType.DMA,
            pltpu.SemaphoreType.DMA,
            pltpu.SemaphoreType.DMA,
        ],
    )(_fuse_body)
    return run(ids_flat, positions, slot_scores)
